# baseline (speedup 1.0000x reference)
import functools

import jax
import jax.numpy as jnp
from jax import lax
from jax.experimental import pallas as pl
from jax.experimental.pallas import tpu as pltpu
from jax.experimental.pallas import tpu_sc as plsc

NC = 2
NS = 16
L = 16

N_USERS = 25000
N_ITEMS = 25000
N_NODES = N_USERS + N_ITEMS
D = 64
DH = D // 2
N_LAYERS = 3
BATCH_B = 4096

E = 800000
CHUNK = 128
UNROLL = 4
N_GROUPS = 100
CHUNKS_PER_TILE = UNROLL * N_GROUPS
E_PAD = NS * CHUNKS_PER_TILE * CHUNK
E_ALLOC = E_PAD + 8 * CHUNK
ROWS_PER_TILE = 3128
ROWS_LAST = N_NODES - (NS - 1) * ROWS_PER_TILE

_mesh = plsc.VectorSubcoreMesh(core_axis_name="c", subcore_axis_name="s")
_sc_params = pltpu.CompilerParams(use_tc_tiling_on_sc=False,
                                 needs_layout_passes=False)


def _layer_body(tab32, src_h, dst_h, val_h, zeros_h, raw_h,
                acc, src_big, dst_big, val_big,
                src2_0, src2_1, src2_2, src2_3,
                dst2_0, dst2_1, dst2_2, dst2_3,
                rows_0, rows_1, rows_2, rows_3,
                ctr_0, ctr_1, ctr_2, ctr_3,
                gsem_0, gsem_1, gsem_2, gsem_3,
                ssem_0, ssem_1, ssem_2, ssem_3, isem):
    c = lax.axis_index("c")
    s = lax.axis_index("s")

    @pl.when(s < NS - 1)
    def _():
        pltpu.sync_copy(zeros_h,
                        acc.at[pl.ds(s * ROWS_PER_TILE, ROWS_PER_TILE)])

    @pl.when(s == NS - 1)
    def _():
        pltpu.sync_copy(zeros_h.at[pl.ds(0, ROWS_LAST)],
                        acc.at[pl.ds((NS - 1) * ROWS_PER_TILE, ROWS_LAST)])

    plsc.subcore_barrier()

    tile_row0 = s * CHUNKS_PER_TILE
    src2 = (src2_0, src2_1, src2_2, src2_3)
    dst2 = (dst2_0, dst2_1, dst2_2, dst2_3)
    rows = (rows_0, rows_1, rows_2, rows_3)
    ctr = (ctr_0, ctr_1, ctr_2, ctr_3)
    gsem = (gsem_0, gsem_1, gsem_2, gsem_3)
    ssem = (ssem_0, ssem_1, ssem_2, ssem_3)

    def issue_load(hrow, brow):
        for h, b in ((src_h, src_big), (dst_h, dst_big), (val_h, val_big)):
            pltpu.async_copy(h.at[pl.ds(hrow, UNROLL), :],
                             b.at[pl.ds(brow, UNROLL), :], isem)

    def wait_load():
        for h, b in ((src_h, src_big), (dst_h, dst_big), (val_h, val_big)):
            pltpu.make_async_copy(h.at[pl.ds(0, UNROLL), :],
                                  b.at[pl.ds(0, UNROLL), :], isem).wait()

    def compute_src2(q, row):
        for k in range(CHUNK // L):
            sl = pl.ds(k * L, L)
            src2[q][sl] = src_big[row, sl] * 2 + c

    def issue_gather(q):
        pltpu.async_copy(tab32.at[src2[q]], rows[q], gsem[q])

    def wait_gather(q):
        pltpu.make_async_copy(tab32.at[src2[q]], rows[q], gsem[q]).wait()

    def scale_and_scatter(p, row):
        himask = jnp.uint32(0xFFFF0000)
        for g in range(CHUNK // L):
            vv16 = val_big[row, pl.ds(g * L, L)]
            for e in range(L):
                r = g * L + e
                vv = jnp.full((L,), vv16[e], jnp.float32)
                u = plsc.bitcast(rows[p][r, pl.ds(0, 2 * L)], jnp.uint32)
                lo = plsc.bitcast(u << 16, jnp.float32)
                hi = plsc.bitcast(u & himask, jnp.float32)
                ctr[p][r, pl.ds(0, L)] = lo * vv
                ctr[p][r, pl.ds(L, L)] = hi * vv
        for k in range(CHUNK // L):
            sl = pl.ds(k * L, L)
            dst2[p][sl] = dst_big[row, sl]
        pltpu.async_copy(ctr[p], acc.at[dst2[p]], ssem[p], add=True)

    def wait_scatter(p):
        pltpu.make_async_copy(ctr[p], acc.at[dst2[p]], ssem[p]).wait()

    issue_load(tile_row0, 0)
    wait_load()
    for j in range(3):
        compute_src2(j, j)
        issue_gather(j)

    def body(t, carry):
        h0 = (t % 2) * UNROLL
        h1 = ((t + 1) % 2) * UNROLL
        issue_load(tile_row0 + (t + 1) * UNROLL, h1)
        for j in range(UNROLL):
            sj = (j + 3) % 4
            wait_gather(j)
            scale_and_scatter(j, h0 + j)
            if j == 1:
                wait_load()

            if j == 0:
                @pl.when(t > 0)
                def _():
                    wait_scatter(sj)
            else:
                wait_scatter(sj)
            if j == 0:
                compute_src2(sj, h0 + 3)
            else:
                compute_src2(sj, h1 + j - 1)
            issue_gather(sj)
        return carry

    lax.fori_loop(0, N_GROUPS, body, 0)

    wait_gather(0)
    wait_gather(1)
    wait_gather(2)
    wait_scatter(3)
    plsc.subcore_barrier()

    ro = s * ROWS_PER_TILE

    @pl.when(s < NS - 1)
    def _():
        pltpu.sync_copy(acc.at[pl.ds(ro, ROWS_PER_TILE)],
                        raw_h.at[pl.ds(c * N_NODES + ro, ROWS_PER_TILE)])

    @pl.when(s == NS - 1)
    def _():
        ro_l = (NS - 1) * ROWS_PER_TILE
        pltpu.sync_copy(acc.at[pl.ds(ro_l, ROWS_LAST)],
                        raw_h.at[pl.ds(c * N_NODES + ro_l, ROWS_LAST)])


_layer_call = pl.kernel(
    _layer_body,
    out_type=jax.ShapeDtypeStruct((NC * N_NODES, DH), jnp.float32),
    mesh=_mesh,
    compiler_params=_sc_params,
    scratch_types=[
        pltpu.VMEM_SHARED((N_NODES, DH), jnp.float32),
        pltpu.VMEM((2 * UNROLL, CHUNK), jnp.int32),
        pltpu.VMEM((2 * UNROLL, CHUNK), jnp.int32),
        pltpu.VMEM((2 * UNROLL, CHUNK), jnp.float32),
        pltpu.VMEM((CHUNK,), jnp.int32),
        pltpu.VMEM((CHUNK,), jnp.int32),
        pltpu.VMEM((CHUNK,), jnp.int32),
        pltpu.VMEM((CHUNK,), jnp.int32),
        pltpu.VMEM((CHUNK,), jnp.int32),
        pltpu.VMEM((CHUNK,), jnp.int32),
        pltpu.VMEM((CHUNK,), jnp.int32),
        pltpu.VMEM((CHUNK,), jnp.int32),
        pltpu.VMEM((CHUNK, DH), jnp.bfloat16),
        pltpu.VMEM((CHUNK, DH), jnp.bfloat16),
        pltpu.VMEM((CHUNK, DH), jnp.bfloat16),
        pltpu.VMEM((CHUNK, DH), jnp.bfloat16),
        pltpu.VMEM((CHUNK, DH), jnp.float32),
        pltpu.VMEM((CHUNK, DH), jnp.float32),
        pltpu.VMEM((CHUNK, DH), jnp.float32),
        pltpu.VMEM((CHUNK, DH), jnp.float32),
        pltpu.SemaphoreType.DMA,
        pltpu.SemaphoreType.DMA,
        pltpu.SemaphoreType.DMA,
        pltpu.SemaphoreType.DMA,
        pltpu.SemaphoreType.DMA,
        pltpu.SemaphoreType.DMA,
        pltpu.SemaphoreType.DMA,
        pltpu.SemaphoreType.DMA,
        pltpu.SemaphoreType.DMA,
    ],
)


def _interleave_halves(full):
    rr = full.shape[0]
    a = full[:, 0:16]
    b = full[:, 16:32]
    cc = full[:, 32:48]
    dd = full[:, 48:64]
    il_ab = jnp.stack([a, b], axis=-1).reshape(rr, 32)
    il_cd = jnp.stack([cc, dd], axis=-1).reshape(rr, 32)
    return jnp.stack([il_ab, il_cd], axis=1).reshape(2 * rr, 32) \
        .astype(jnp.bfloat16)


def _norm_body(scale, raw_ref, accin_ref, tabbf_ref, accout_ref):
    ra = raw_ref[0]
    rb = raw_ref[1]
    ss = (jnp.sum(ra * ra, axis=1, keepdims=True)
          + jnp.sum(rb * rb, axis=1, keepdims=True))
    r = lax.rsqrt(jnp.maximum(ss, 1e-12))
    full = jnp.concatenate([ra * r, rb * r], axis=1)
    tabbf_ref[...] = _interleave_halves(full)
    accout_ref[...] = scale * accin_ref[...] + 0.25 * full


_NORM_ROWS = 1000


def _make_norm_call(scale):
    return pl.pallas_call(
        functools.partial(_norm_body, scale),
        grid=(N_NODES // _NORM_ROWS,),
        in_specs=[
            pl.BlockSpec((NC, _NORM_ROWS, DH), lambda i: (0, i, 0)),
            pl.BlockSpec((_NORM_ROWS, D), lambda i: (i, 0)),
        ],
        out_specs=[
            pl.BlockSpec((NC * _NORM_ROWS, DH), lambda i: (i, 0)),
            pl.BlockSpec((_NORM_ROWS, D), lambda i: (i, 0)),
        ],
        out_shape=[
            jax.ShapeDtypeStruct((NC * N_NODES, DH), jnp.bfloat16),
            jax.ShapeDtypeStruct((N_NODES, D), jnp.float32),
        ],
    )


_norm_first = _make_norm_call(0.25)
_norm_rest = _make_norm_call(1.0)


_B_PER_TILE = BATCH_B // (NC * NS)


def _gather_body(light, users_h, pos_h, neg_h, u_out, p_out, n_out,
                 u_idx, p_idx, n_idx, rows_u, rows_p, rows_n, sem):
    c = lax.axis_index("c")
    s = lax.axis_index("s")
    wid = s * NC + c
    base = wid * _B_PER_TILE

    pltpu.sync_copy(users_h.at[pl.ds(base, _B_PER_TILE)], u_idx)
    pltpu.sync_copy(pos_h.at[pl.ds(base, _B_PER_TILE)], p_idx)
    pltpu.sync_copy(neg_h.at[pl.ds(base, _B_PER_TILE)], n_idx)

    for k in range(_B_PER_TILE // L):
        sl = pl.ds(k * L, L)
        p_idx[sl] = p_idx[sl] + N_USERS
        n_idx[sl] = n_idx[sl] + N_USERS

    pltpu.async_copy(light.at[u_idx], rows_u, sem).wait()
    pltpu.async_copy(light.at[p_idx], rows_p, sem).wait()
    pltpu.async_copy(light.at[n_idx], rows_n, sem).wait()

    pltpu.sync_copy(rows_u, u_out.at[pl.ds(base, _B_PER_TILE)])
    pltpu.sync_copy(rows_p, p_out.at[pl.ds(base, _B_PER_TILE)])
    pltpu.sync_copy(rows_n, n_out.at[pl.ds(base, _B_PER_TILE)])


_gather_call = pl.kernel(
    _gather_body,
    out_type=[
        jax.ShapeDtypeStruct((BATCH_B, D), jnp.float32),
        jax.ShapeDtypeStruct((BATCH_B, D), jnp.float32),
        jax.ShapeDtypeStruct((BATCH_B, D), jnp.float32),
    ],
    mesh=_mesh,
    compiler_params=_sc_params,
    scratch_types=[
        pltpu.VMEM((_B_PER_TILE,), jnp.int32),
        pltpu.VMEM((_B_PER_TILE,), jnp.int32),
        pltpu.VMEM((_B_PER_TILE,), jnp.int32),
        pltpu.VMEM((_B_PER_TILE, D), jnp.float32),
        pltpu.VMEM((_B_PER_TILE, D), jnp.float32),
        pltpu.VMEM((_B_PER_TILE, D), jnp.float32),
        pltpu.SemaphoreType.DMA,
    ],
)


def _dot_body(u_ref, p_ref, n_ref, ps_ref, ns_ref):
    u = u_ref[...]
    ps_ref[...] = jnp.sum(u * p_ref[...], axis=1)
    ns_ref[...] = jnp.sum(u * n_ref[...], axis=1)


_dot_call = pl.pallas_call(
    _dot_body,
    out_shape=[
        jax.ShapeDtypeStruct((BATCH_B,), jnp.float32),
        jax.ShapeDtypeStruct((BATCH_B,), jnp.float32),
    ],
)


def kernel(users, pos_items, neg_items, adj_indices, adj_values, user_table,
           item_table):
    users = users.astype(jnp.int32)
    pos_items = pos_items.astype(jnp.int32)
    neg_items = neg_items.astype(jnp.int32)

    dst = adj_indices[0].astype(jnp.int32)
    src = adj_indices[1].astype(jnp.int32)
    vals = adj_values.astype(jnp.float32)

    pad = E_ALLOC - E
    src_p = jnp.concatenate([src, jnp.zeros((pad,), jnp.int32)]) \
        .reshape(E_ALLOC // CHUNK, CHUNK)
    dst_p = jnp.concatenate([dst, jnp.zeros((pad,), jnp.int32)]) \
        .reshape(E_ALLOC // CHUNK, CHUNK)
    val_p = jnp.concatenate([vals, jnp.zeros((pad,), jnp.float32)]) \
        .reshape(E_ALLOC // CHUNK, CHUNK)

    all_emb = jnp.concatenate([user_table, item_table], axis=0)
    zeros = jnp.zeros((ROWS_PER_TILE, DH), jnp.float32)

    tab_bf = _interleave_halves(all_emb)
    acc = all_emb
    for layer in range(N_LAYERS):
        raw = _layer_call(tab_bf, src_p, dst_p, val_p, zeros)
        norm_call = _norm_first if layer == 0 else _norm_rest
        tab_bf, acc = norm_call(raw.reshape(NC, N_NODES, DH), acc)

    u_rows, p_rows, n_rows = _gather_call(acc, users, pos_items, neg_items)
    pos_scores, neg_scores = _dot_call(u_rows, p_rows, n_rows)
    return (pos_scores, neg_scores, acc[:N_USERS], acc[N_USERS:])


# speedup vs baseline: 4.3897x; 4.3897x over previous
import functools

import jax
import jax.numpy as jnp
from jax import lax
from jax.experimental import pallas as pl
from jax.experimental.pallas import tpu as pltpu
from jax.experimental.pallas import tpu_sc as plsc

NC = 2
NS = 16
L = 16

N_USERS = 25000
N_ITEMS = 25000
N_NODES = N_USERS + N_ITEMS
D = 64
DH = D // 2
N_LAYERS = 3
BATCH_B = 4096

E = 800000
CHUNK = 128
UNROLL = 4
N_GROUPS = 100
CHUNKS_PER_TILE = UNROLL * N_GROUPS
E_PAD = NS * CHUNKS_PER_TILE * CHUNK
E_ALLOC = E_PAD + 8 * CHUNK
ROWS_PER_TILE = 3128
ROWS_LAST = N_NODES - (NS - 1) * ROWS_PER_TILE

_mesh = plsc.VectorSubcoreMesh(core_axis_name="c", subcore_axis_name="s")
_sc_params = pltpu.CompilerParams(use_tc_tiling_on_sc=False)


def _layer_body(tab32, src_h, dst_h, val_h, zeros_h, raw_h,
                acc, src_big, dst_big, val_big,
                src2_0, src2_1, src2_2, src2_3,
                dst2_0, dst2_1, dst2_2, dst2_3,
                rows_0, rows_1, rows_2, rows_3,
                ctr_0, ctr_1, ctr_2, ctr_3,
                gsem_0, gsem_1, gsem_2, gsem_3,
                ssem_0, ssem_1, ssem_2, ssem_3, isem):
    c = lax.axis_index("c")
    s = lax.axis_index("s")

    @pl.when(s < NS - 1)
    def _():
        pltpu.sync_copy(zeros_h,
                        acc.at[pl.ds(s * ROWS_PER_TILE, ROWS_PER_TILE)])

    @pl.when(s == NS - 1)
    def _():
        pltpu.sync_copy(zeros_h.at[pl.ds(0, ROWS_LAST)],
                        acc.at[pl.ds((NS - 1) * ROWS_PER_TILE, ROWS_LAST)])

    plsc.subcore_barrier()

    tile_row0 = s * CHUNKS_PER_TILE
    src2 = (src2_0, src2_1, src2_2, src2_3)
    dst2 = (dst2_0, dst2_1, dst2_2, dst2_3)
    rows = (rows_0, rows_1, rows_2, rows_3)
    ctr = (ctr_0, ctr_1, ctr_2, ctr_3)
    gsem = (gsem_0, gsem_1, gsem_2, gsem_3)
    ssem = (ssem_0, ssem_1, ssem_2, ssem_3)

    def issue_load(hrow, brow):
        for h, b in ((src_h, src_big), (dst_h, dst_big), (val_h, val_big)):
            pltpu.async_copy(h.at[pl.ds(hrow, UNROLL), :],
                             b.at[pl.ds(brow, UNROLL), :], isem)

    def wait_load():
        for h, b in ((src_h, src_big), (dst_h, dst_big), (val_h, val_big)):
            pltpu.make_async_copy(h.at[pl.ds(0, UNROLL), :],
                                  b.at[pl.ds(0, UNROLL), :], isem).wait()

    def compute_src2(q, row):
        for k in range(CHUNK // L):
            sl = pl.ds(k * L, L)
            src2[q][sl] = src_big[row, sl] * 2 + c

    def issue_gather(q):
        pltpu.async_copy(tab32.at[src2[q]], rows[q], gsem[q])

    def wait_gather(q):
        pltpu.make_async_copy(tab32.at[src2[q]], rows[q], gsem[q]).wait()

    def scale_and_scatter(p, row):
        himask = jnp.uint32(0xFFFF0000)
        for g in range(CHUNK // L):
            vv16 = val_big[row, pl.ds(g * L, L)]
            for e in range(L):
                r = g * L + e
                vv = jnp.full((L,), vv16[e], jnp.float32)
                u = rows[p][r, pl.ds(0, L)]
                lo = lax.bitcast_convert_type(u << 16, jnp.float32)
                hi = lax.bitcast_convert_type(u & himask, jnp.float32)
                ctr[p][r, pl.ds(0, L)] = lo * vv
                ctr[p][r, pl.ds(L, L)] = hi * vv
        for k in range(CHUNK // L):
            sl = pl.ds(k * L, L)
            dst2[p][sl] = dst_big[row, sl]
        pltpu.async_copy(ctr[p], acc.at[dst2[p]], ssem[p], add=True)

    def wait_scatter(p):
        pltpu.make_async_copy(ctr[p], acc.at[dst2[p]], ssem[p]).wait()

    issue_load(tile_row0, 0)
    wait_load()
    for j in range(3):
        compute_src2(j, j)
        issue_gather(j)

    def body(t, carry):
        h0 = (t % 2) * UNROLL
        h1 = ((t + 1) % 2) * UNROLL
        issue_load(tile_row0 + (t + 1) * UNROLL, h1)
        for j in range(UNROLL):
            sj = (j + 3) % 4
            wait_gather(j)
            scale_and_scatter(j, h0 + j)
            if j == 1:
                wait_load()

            if j == 0:
                @pl.when(t > 0)
                def _():
                    wait_scatter(sj)
            else:
                wait_scatter(sj)
            if j == 0:
                compute_src2(sj, h0 + 3)
            else:
                compute_src2(sj, h1 + j - 1)
            issue_gather(sj)
        return carry

    lax.fori_loop(0, N_GROUPS, body, 0)

    wait_gather(0)
    wait_gather(1)
    wait_gather(2)
    wait_scatter(3)
    plsc.subcore_barrier()

    ro = s * ROWS_PER_TILE

    @pl.when(s < NS - 1)
    def _():
        pltpu.sync_copy(acc.at[pl.ds(ro, ROWS_PER_TILE)],
                        raw_h.at[pl.ds(c * N_NODES + ro, ROWS_PER_TILE)])

    @pl.when(s == NS - 1)
    def _():
        ro_l = (NS - 1) * ROWS_PER_TILE
        pltpu.sync_copy(acc.at[pl.ds(ro_l, ROWS_LAST)],
                        raw_h.at[pl.ds(c * N_NODES + ro_l, ROWS_LAST)])


_layer_call = pl.kernel(
    _layer_body,
    out_type=jax.ShapeDtypeStruct((NC * N_NODES, DH), jnp.float32),
    mesh=_mesh,
    compiler_params=_sc_params,
    scratch_types=[
        pltpu.VMEM_SHARED((N_NODES, DH), jnp.float32),
        pltpu.VMEM((2 * UNROLL, CHUNK), jnp.int32),
        pltpu.VMEM((2 * UNROLL, CHUNK), jnp.int32),
        pltpu.VMEM((2 * UNROLL, CHUNK), jnp.float32),
        pltpu.VMEM((CHUNK,), jnp.int32),
        pltpu.VMEM((CHUNK,), jnp.int32),
        pltpu.VMEM((CHUNK,), jnp.int32),
        pltpu.VMEM((CHUNK,), jnp.int32),
        pltpu.VMEM((CHUNK,), jnp.int32),
        pltpu.VMEM((CHUNK,), jnp.int32),
        pltpu.VMEM((CHUNK,), jnp.int32),
        pltpu.VMEM((CHUNK,), jnp.int32),
        pltpu.VMEM((CHUNK, DH // 2), jnp.uint32),
        pltpu.VMEM((CHUNK, DH // 2), jnp.uint32),
        pltpu.VMEM((CHUNK, DH // 2), jnp.uint32),
        pltpu.VMEM((CHUNK, DH // 2), jnp.uint32),
        pltpu.VMEM((CHUNK, DH), jnp.float32),
        pltpu.VMEM((CHUNK, DH), jnp.float32),
        pltpu.VMEM((CHUNK, DH), jnp.float32),
        pltpu.VMEM((CHUNK, DH), jnp.float32),
        pltpu.SemaphoreType.DMA,
        pltpu.SemaphoreType.DMA,
        pltpu.SemaphoreType.DMA,
        pltpu.SemaphoreType.DMA,
        pltpu.SemaphoreType.DMA,
        pltpu.SemaphoreType.DMA,
        pltpu.SemaphoreType.DMA,
        pltpu.SemaphoreType.DMA,
        pltpu.SemaphoreType.DMA,
    ],
)


def _pack_table(full):
    rr = full.shape[0]

    def rne(x):
        b = lax.bitcast_convert_type(x, jnp.uint32)
        return (b + jnp.uint32(0x7FFF) + ((b >> 16) & jnp.uint32(1))) >> 16

    row_a = rne(full[:, 0:16]) | (rne(full[:, 16:32]) << 16)
    row_b = rne(full[:, 32:48]) | (rne(full[:, 48:64]) << 16)
    return jnp.stack([row_a, row_b], axis=1).reshape(2 * rr, 16)


def _norm_body(scale, raw_ref, accin_ref, tabbf_ref, accout_ref):
    ra = raw_ref[0]
    rb = raw_ref[1]
    ss = (jnp.sum(ra * ra, axis=1, keepdims=True)
          + jnp.sum(rb * rb, axis=1, keepdims=True))
    r = lax.rsqrt(jnp.maximum(ss, 1e-12))
    full = jnp.concatenate([ra * r, rb * r], axis=1)
    tabbf_ref[...] = _pack_table(full)
    accout_ref[...] = scale * accin_ref[...] + 0.25 * full


_NORM_ROWS = 1000


def _make_norm_call(scale):
    return pl.pallas_call(
        functools.partial(_norm_body, scale),
        grid=(N_NODES // _NORM_ROWS,),
        in_specs=[
            pl.BlockSpec((NC, _NORM_ROWS, DH), lambda i: (0, i, 0)),
            pl.BlockSpec((_NORM_ROWS, D), lambda i: (i, 0)),
        ],
        out_specs=[
            pl.BlockSpec((NC * _NORM_ROWS, DH // 2), lambda i: (i, 0)),
            pl.BlockSpec((_NORM_ROWS, D), lambda i: (i, 0)),
        ],
        out_shape=[
            jax.ShapeDtypeStruct((NC * N_NODES, DH // 2), jnp.uint32),
            jax.ShapeDtypeStruct((N_NODES, D), jnp.float32),
        ],
    )


_norm_first = _make_norm_call(0.25)
_norm_rest = _make_norm_call(1.0)


_B_PER_TILE = BATCH_B // (NC * NS)


def _gather_body(light, users_h, pos_h, neg_h, u_out, p_out, n_out,
                 u_idx, p_idx, n_idx, rows_u, rows_p, rows_n, sem):
    c = lax.axis_index("c")
    s = lax.axis_index("s")
    wid = s * NC + c
    base = wid * _B_PER_TILE

    pltpu.sync_copy(users_h.at[pl.ds(base, _B_PER_TILE)], u_idx)
    pltpu.sync_copy(pos_h.at[pl.ds(base, _B_PER_TILE)], p_idx)
    pltpu.sync_copy(neg_h.at[pl.ds(base, _B_PER_TILE)], n_idx)

    for k in range(_B_PER_TILE // L):
        sl = pl.ds(k * L, L)
        p_idx[sl] = p_idx[sl] + N_USERS
        n_idx[sl] = n_idx[sl] + N_USERS

    pltpu.async_copy(light.at[u_idx], rows_u, sem).wait()
    pltpu.async_copy(light.at[p_idx], rows_p, sem).wait()
    pltpu.async_copy(light.at[n_idx], rows_n, sem).wait()

    pltpu.sync_copy(rows_u, u_out.at[pl.ds(base, _B_PER_TILE)])
    pltpu.sync_copy(rows_p, p_out.at[pl.ds(base, _B_PER_TILE)])
    pltpu.sync_copy(rows_n, n_out.at[pl.ds(base, _B_PER_TILE)])


_gather_call = pl.kernel(
    _gather_body,
    out_type=[
        jax.ShapeDtypeStruct((BATCH_B, D), jnp.float32),
        jax.ShapeDtypeStruct((BATCH_B, D), jnp.float32),
        jax.ShapeDtypeStruct((BATCH_B, D), jnp.float32),
    ],
    mesh=_mesh,
    compiler_params=_sc_params,
    scratch_types=[
        pltpu.VMEM((_B_PER_TILE,), jnp.int32),
        pltpu.VMEM((_B_PER_TILE,), jnp.int32),
        pltpu.VMEM((_B_PER_TILE,), jnp.int32),
        pltpu.VMEM((_B_PER_TILE, D), jnp.float32),
        pltpu.VMEM((_B_PER_TILE, D), jnp.float32),
        pltpu.VMEM((_B_PER_TILE, D), jnp.float32),
        pltpu.SemaphoreType.DMA,
    ],
)


def _dot_body(u_ref, p_ref, n_ref, ps_ref, ns_ref):
    u = u_ref[...]
    ps_ref[...] = jnp.sum(u * p_ref[...], axis=1)
    ns_ref[...] = jnp.sum(u * n_ref[...], axis=1)


_dot_call = pl.pallas_call(
    _dot_body,
    out_shape=[
        jax.ShapeDtypeStruct((BATCH_B,), jnp.float32),
        jax.ShapeDtypeStruct((BATCH_B,), jnp.float32),
    ],
)


def kernel(users, pos_items, neg_items, adj_indices, adj_values, user_table,
           item_table):
    users = users.astype(jnp.int32)
    pos_items = pos_items.astype(jnp.int32)
    neg_items = neg_items.astype(jnp.int32)

    dst = adj_indices[0].astype(jnp.int32)
    src = adj_indices[1].astype(jnp.int32)
    vals = adj_values.astype(jnp.float32)

    pad = E_ALLOC - E
    src_p = jnp.concatenate([src, jnp.zeros((pad,), jnp.int32)]) \
        .reshape(E_ALLOC // CHUNK, CHUNK)
    dst_p = jnp.concatenate([dst, jnp.zeros((pad,), jnp.int32)]) \
        .reshape(E_ALLOC // CHUNK, CHUNK)
    val_p = jnp.concatenate([vals, jnp.zeros((pad,), jnp.float32)]) \
        .reshape(E_ALLOC // CHUNK, CHUNK)

    all_emb = jnp.concatenate([user_table, item_table], axis=0)
    zeros = jnp.zeros((ROWS_PER_TILE, DH), jnp.float32)

    tab_bf = _pack_table(all_emb)
    acc = all_emb
    for layer in range(N_LAYERS):
        raw = _layer_call(tab_bf, src_p, dst_p, val_p, zeros)
        norm_call = _norm_first if layer == 0 else _norm_rest
        tab_bf, acc = norm_call(raw.reshape(NC, N_NODES, DH), acc)

    u_rows, p_rows, n_rows = _gather_call(acc, users, pos_items, neg_items)
    pos_scores, neg_scores = _dot_call(u_rows, p_rows, n_rows)
    return (pos_scores, neg_scores, acc[:N_USERS], acc[N_USERS:])


# speedup vs baseline: 5.0810x; 1.1575x over previous
import functools

import jax
import jax.numpy as jnp
from jax import lax
from jax.experimental import pallas as pl
from jax.experimental.pallas import tpu as pltpu
from jax.experimental.pallas import tpu_sc as plsc

NC = 2
NS = 16
L = 16

N_USERS = 25000
N_ITEMS = 25000
N_NODES = N_USERS + N_ITEMS
D = 64
DP = 16
N_LAYERS = 3
BATCH_B = 4096

E = 800000
CHUNK = 128
UNROLL = 4
N_GROUPS = 100
CHUNKS_PER_TILE = UNROLL * N_GROUPS
E_PAD = NS * CHUNKS_PER_TILE * CHUNK
E_ALLOC = E_PAD + 8 * CHUNK
ROWS_PER_TILE = 3128
ROWS_LAST = N_NODES - (NS - 1) * ROWS_PER_TILE

_mesh = plsc.VectorSubcoreMesh(core_axis_name="c", subcore_axis_name="s")
_sc_params = pltpu.CompilerParams(use_tc_tiling_on_sc=False)


def _pass_body(tab_h, src_h, dst_h, val_h, zeros_h, raw_h,
               tab_sp, acc_sp,
               dst2_0, dst2_1, dst2_2, dst2_3,
               rows_0, rows_1, rows_2, rows_3,
               src_big, dst_big, val_big,
               gsem_0, gsem_1, gsem_2, gsem_3,
               ssem_0, ssem_1, ssem_2, ssem_3, isem):
    c = lax.axis_index("c")
    s = lax.axis_index("s")

    ro = s * ROWS_PER_TILE

    @pl.when(s < NS - 1)
    def _():
        pltpu.sync_copy(zeros_h, acc_sp.at[pl.ds(ro, ROWS_PER_TILE)])
        pltpu.sync_copy(tab_h.at[pl.ds(c * N_NODES + ro, ROWS_PER_TILE)],
                        tab_sp.at[pl.ds(ro, ROWS_PER_TILE)])

    @pl.when(s == NS - 1)
    def _():
        ro_l = (NS - 1) * ROWS_PER_TILE
        pltpu.sync_copy(zeros_h.at[pl.ds(0, ROWS_LAST)],
                        acc_sp.at[pl.ds(ro_l, ROWS_LAST)])
        pltpu.sync_copy(tab_h.at[pl.ds(c * N_NODES + ro_l, ROWS_LAST)],
                        tab_sp.at[pl.ds(ro_l, ROWS_LAST)])

    plsc.subcore_barrier()

    tile_row0 = s * CHUNKS_PER_TILE
    dst2 = (dst2_0, dst2_1, dst2_2, dst2_3)
    rows = (rows_0, rows_1, rows_2, rows_3)
    gsem = (gsem_0, gsem_1, gsem_2, gsem_3)
    ssem = (ssem_0, ssem_1, ssem_2, ssem_3)

    def issue_load(hrow, brow):
        for h, b in ((src_h, src_big), (dst_h, dst_big), (val_h, val_big)):
            pltpu.async_copy(h.at[pl.ds(hrow, UNROLL), :],
                             b.at[pl.ds(brow, UNROLL), :], isem)

    def wait_load():
        for h, b in ((src_h, src_big), (dst_h, dst_big), (val_h, val_big)):
            pltpu.make_async_copy(h.at[pl.ds(0, UNROLL), :],
                                  b.at[pl.ds(0, UNROLL), :], isem).wait()

    def issue_gather(q, row):
        pltpu.async_copy(tab_sp.at[src_big.at[row]], rows[q], gsem[q])

    def wait_gather(q, row):
        pltpu.make_async_copy(tab_sp.at[src_big.at[row]], rows[q],
                              gsem[q]).wait()

    def scale_and_scatter(p, row):
        for g in range(CHUNK // L):
            vv16 = val_big[row, pl.ds(g * L, L)]
            for e in range(L):
                r = g * L + e
                vv = jnp.full((L,), vv16[e], jnp.float32)
                rows[p][r, pl.ds(0, L)] = rows[p][r, pl.ds(0, L)] * vv
        for k in range(CHUNK // L):
            sl = pl.ds(k * L, L)
            dst2[p][sl] = dst_big[row, sl]
        pltpu.async_copy(rows[p], acc_sp.at[dst2[p]], ssem[p], add=True)

    def wait_scatter(p):
        pltpu.make_async_copy(rows[p], acc_sp.at[dst2[p]], ssem[p]).wait()

    issue_load(tile_row0, 0)
    wait_load()
    for j in range(3):
        issue_gather(j, j)

    def body(t, carry):
        h0 = (t % 2) * UNROLL
        h1 = ((t + 1) % 2) * UNROLL
        issue_load(tile_row0 + (t + 1) * UNROLL, h1)
        for j in range(UNROLL):
            sj = (j + 3) % 4
            row = h0 + j
            wait_gather(j, row)
            scale_and_scatter(j, row)
            if j == 1:
                wait_load()

            if j == 0:
                @pl.when(t > 0)
                def _():
                    wait_scatter(sj)
            else:
                wait_scatter(sj)
            nrow = h0 + 3 if j == 0 else h1 + j - 1
            issue_gather(sj, nrow)
        return carry

    lax.fori_loop(0, N_GROUPS, body, 0)

    wait_gather(0, 0)
    wait_gather(1, 1)
    wait_gather(2, 2)
    wait_scatter(3)
    plsc.subcore_barrier()

    @pl.when(s < NS - 1)
    def _():
        pltpu.sync_copy(acc_sp.at[pl.ds(ro, ROWS_PER_TILE)],
                        raw_h.at[pl.ds(c * N_NODES + ro, ROWS_PER_TILE)])

    @pl.when(s == NS - 1)
    def _():
        ro_l = (NS - 1) * ROWS_PER_TILE
        pltpu.sync_copy(acc_sp.at[pl.ds(ro_l, ROWS_LAST)],
                        raw_h.at[pl.ds(c * N_NODES + ro_l, ROWS_LAST)])


_pass_call = pl.kernel(
    _pass_body,
    out_type=jax.ShapeDtypeStruct((NC * N_NODES, DP), jnp.float32),
    mesh=_mesh,
    compiler_params=_sc_params,
    scratch_types=[
        pltpu.VMEM_SHARED((N_NODES, DP), jnp.float32),
        pltpu.VMEM_SHARED((N_NODES, DP), jnp.float32),
        pltpu.VMEM((CHUNK,), jnp.int32),
        pltpu.VMEM((CHUNK,), jnp.int32),
        pltpu.VMEM((CHUNK,), jnp.int32),
        pltpu.VMEM((CHUNK,), jnp.int32),
        pltpu.VMEM((CHUNK, DP), jnp.float32),
        pltpu.VMEM((CHUNK, DP), jnp.float32),
        pltpu.VMEM((CHUNK, DP), jnp.float32),
        pltpu.VMEM((CHUNK, DP), jnp.float32),
        pltpu.VMEM((2 * UNROLL, CHUNK), jnp.int32),
        pltpu.VMEM((2 * UNROLL, CHUNK), jnp.int32),
        pltpu.VMEM((2 * UNROLL, CHUNK), jnp.float32),
        pltpu.SemaphoreType.DMA,
        pltpu.SemaphoreType.DMA,
        pltpu.SemaphoreType.DMA,
        pltpu.SemaphoreType.DMA,
        pltpu.SemaphoreType.DMA,
        pltpu.SemaphoreType.DMA,
        pltpu.SemaphoreType.DMA,
        pltpu.SemaphoreType.DMA,
        pltpu.SemaphoreType.DMA,
    ],
)


def _norm_body(scale, rawa_ref, rawb_ref, accin_ref, taba_ref, tabb_ref,
               accout_ref):
    full = jnp.concatenate(
        [rawa_ref[0], rawb_ref[0], rawa_ref[1], rawb_ref[1]], axis=1)
    ss = jnp.sum(full * full, axis=1, keepdims=True)
    r = lax.rsqrt(jnp.maximum(ss, 1e-12))
    full = full * r
    taba_ref[...] = jnp.stack([full[:, 0:16], full[:, 32:48]], axis=0)
    tabb_ref[...] = jnp.stack([full[:, 16:32], full[:, 48:64]], axis=0)
    accout_ref[...] = scale * accin_ref[...] + 0.25 * full


_NORM_ROWS = 1000


def _make_norm_call(scale):
    return pl.pallas_call(
        functools.partial(_norm_body, scale),
        grid=(N_NODES // _NORM_ROWS,),
        in_specs=[
            pl.BlockSpec((NC, _NORM_ROWS, DP), lambda i: (0, i, 0)),
            pl.BlockSpec((NC, _NORM_ROWS, DP), lambda i: (0, i, 0)),
            pl.BlockSpec((_NORM_ROWS, D), lambda i: (i, 0)),
        ],
        out_specs=[
            pl.BlockSpec((NC, _NORM_ROWS, DP), lambda i: (0, i, 0)),
            pl.BlockSpec((NC, _NORM_ROWS, DP), lambda i: (0, i, 0)),
            pl.BlockSpec((_NORM_ROWS, D), lambda i: (i, 0)),
        ],
        out_shape=[
            jax.ShapeDtypeStruct((NC, N_NODES, DP), jnp.float32),
            jax.ShapeDtypeStruct((NC, N_NODES, DP), jnp.float32),
            jax.ShapeDtypeStruct((N_NODES, D), jnp.float32),
        ],
    )


_norm_first = _make_norm_call(0.25)
_norm_rest = _make_norm_call(1.0)


_B_PER_TILE = BATCH_B // (NC * NS)


def _gather_body(light, users_h, pos_h, neg_h, u_out, p_out, n_out,
                 u_idx, p_idx, n_idx, rows_u, rows_p, rows_n, sem):
    c = lax.axis_index("c")
    s = lax.axis_index("s")
    wid = s * NC + c
    base = wid * _B_PER_TILE

    pltpu.sync_copy(users_h.at[pl.ds(base, _B_PER_TILE)], u_idx)
    pltpu.sync_copy(pos_h.at[pl.ds(base, _B_PER_TILE)], p_idx)
    pltpu.sync_copy(neg_h.at[pl.ds(base, _B_PER_TILE)], n_idx)

    for k in range(_B_PER_TILE // L):
        sl = pl.ds(k * L, L)
        p_idx[sl] = p_idx[sl] + N_USERS
        n_idx[sl] = n_idx[sl] + N_USERS

    pltpu.async_copy(light.at[u_idx], rows_u, sem).wait()
    pltpu.async_copy(light.at[p_idx], rows_p, sem).wait()
    pltpu.async_copy(light.at[n_idx], rows_n, sem).wait()

    pltpu.sync_copy(rows_u, u_out.at[pl.ds(base, _B_PER_TILE)])
    pltpu.sync_copy(rows_p, p_out.at[pl.ds(base, _B_PER_TILE)])
    pltpu.sync_copy(rows_n, n_out.at[pl.ds(base, _B_PER_TILE)])


_gather_call = pl.kernel(
    _gather_body,
    out_type=[
        jax.ShapeDtypeStruct((BATCH_B, D), jnp.float32),
        jax.ShapeDtypeStruct((BATCH_B, D), jnp.float32),
        jax.ShapeDtypeStruct((BATCH_B, D), jnp.float32),
    ],
    mesh=_mesh,
    compiler_params=_sc_params,
    scratch_types=[
        pltpu.VMEM((_B_PER_TILE,), jnp.int32),
        pltpu.VMEM((_B_PER_TILE,), jnp.int32),
        pltpu.VMEM((_B_PER_TILE,), jnp.int32),
        pltpu.VMEM((_B_PER_TILE, D), jnp.float32),
        pltpu.VMEM((_B_PER_TILE, D), jnp.float32),
        pltpu.VMEM((_B_PER_TILE, D), jnp.float32),
        pltpu.SemaphoreType.DMA,
    ],
)


def _dot_body(u_ref, p_ref, n_ref, ps_ref, ns_ref):
    u = u_ref[...]
    ps_ref[...] = jnp.sum(u * p_ref[...], axis=1)
    ns_ref[...] = jnp.sum(u * n_ref[...], axis=1)


_dot_call = pl.pallas_call(
    _dot_body,
    out_shape=[
        jax.ShapeDtypeStruct((BATCH_B,), jnp.float32),
        jax.ShapeDtypeStruct((BATCH_B,), jnp.float32),
    ],
)


def kernel(users, pos_items, neg_items, adj_indices, adj_values, user_table,
           item_table):
    users = users.astype(jnp.int32)
    pos_items = pos_items.astype(jnp.int32)
    neg_items = neg_items.astype(jnp.int32)

    dst = adj_indices[0].astype(jnp.int32)
    src = adj_indices[1].astype(jnp.int32)
    vals = adj_values.astype(jnp.float32)

    pad = E_ALLOC - E
    src_p = jnp.concatenate([src, jnp.zeros((pad,), jnp.int32)]) \
        .reshape(E_ALLOC // CHUNK, CHUNK)
    dst_p = jnp.concatenate([dst, jnp.zeros((pad,), jnp.int32)]) \
        .reshape(E_ALLOC // CHUNK, CHUNK)
    val_p = jnp.concatenate([vals, jnp.zeros((pad,), jnp.float32)]) \
        .reshape(E_ALLOC // CHUNK, CHUNK)

    all_emb = jnp.concatenate([user_table, item_table], axis=0)
    zeros = jnp.zeros((ROWS_PER_TILE, DP), jnp.float32)

    tab_a = jnp.stack([all_emb[:, 0:16], all_emb[:, 32:48]], axis=0) \
        .reshape(NC * N_NODES, DP)
    tab_b = jnp.stack([all_emb[:, 16:32], all_emb[:, 48:64]], axis=0) \
        .reshape(NC * N_NODES, DP)
    acc = all_emb
    for layer in range(N_LAYERS):
        raw_a = _pass_call(tab_a, src_p, dst_p, val_p, zeros)
        raw_b = _pass_call(tab_b, src_p, dst_p, val_p, zeros)
        norm_call = _norm_first if layer == 0 else _norm_rest
        tab_a, tab_b, acc = norm_call(raw_a.reshape(NC, N_NODES, DP),
                                      raw_b.reshape(NC, N_NODES, DP), acc)
        tab_a = tab_a.reshape(NC * N_NODES, DP)
        tab_b = tab_b.reshape(NC * N_NODES, DP)

    u_rows, p_rows, n_rows = _gather_call(acc, users, pos_items, neg_items)
    pos_scores, neg_scores = _dot_call(u_rows, p_rows, n_rows)
    return (pos_scores, neg_scores, acc[:N_USERS], acc[N_USERS:])
